# baseline (speedup 1.0000x reference)
import functools

import jax
import jax.numpy as jnp
from jax import lax
from jax.experimental import pallas as pl
from jax.experimental.pallas import tpu as pltpu
from jax.experimental.pallas import tpu_sc as plsc

_NC = 2
_NS = 16
_NW = _NC * _NS
_L = 16

_GB = 4096


def _rne_bf16_bits(x):
    b = lax.bitcast_convert_type(x, jnp.int32)
    r = b + 0x7FFF + (lax.shift_right_logical(b, 16) & 1)
    return lax.shift_right_logical(r, 16)


def _pack8(s):
    rows = [
        _rne_bf16_bits(s[:, 512 * j:512 * j + 256])
        | (_rne_bf16_bits(s[:, 512 * j + 256:512 * j + 512]) << 16)
        for j in range(8)
    ]
    return jnp.concatenate(rows, axis=0)


def _encode_body(ax_ref, px_ref, va_ref, vp_ref, c_ref, oa_ref, op_ref):
    asum = jnp.sum(ax_ref[...] * va_ref[...], axis=0, keepdims=True)
    oa_ref[...] = _pack8(asum)
    psum = jnp.sum(px_ref[...] * vp_ref[...], axis=0, keepdims=True) + c_ref[0]
    op_ref[...] = _pack8(psum)


def _encode(axt, pxt, va, vp, const):
    d, na = axt.shape
    npp = pxt.shape[1]
    grid = -(-max(na, npp) // _GB)
    nba = -(-na // _GB) - 1
    nbp = -(-npp // _GB) - 1

    return pl.pallas_call(
        _encode_body,
        grid=(grid,),
        out_shape=[
            jax.ShapeDtypeStruct((8 * grid, 256), jnp.int32),
            jax.ShapeDtypeStruct((8 * grid, 256), jnp.int32),
        ],
        in_specs=[
            pl.BlockSpec((d, _GB), lambda g: (0, jnp.minimum(g, nba))),
            pl.BlockSpec((d, _GB), lambda g: (0, jnp.minimum(g, nbp))),
            pl.BlockSpec((d, 1), lambda g: (0, 0)),
            pl.BlockSpec((d, 1), lambda g: (0, 0)),
            pl.BlockSpec(memory_space=pltpu.SMEM),
        ],
        out_specs=[
            pl.BlockSpec((8, 256), lambda g: (g, 0)),
            pl.BlockSpec((8, 256), lambda g: (g, 0)),
        ],
    )(axt, pxt, va, vp, const)


_CROWS = 32
_KE = _CROWS * 128


def _make_edge_kernel(rows_total, ta_rows, tp_rows):
    rows8 = rows_total // 8
    base8 = rows8 // _NW
    extra8 = rows8 - base8 * _NW
    chunks = -(-(base8 + 1) * 8 // _CROWS)
    steps = _KE // _L
    mesh = plsc.VectorSubcoreMesh(
        core_axis_name="c", subcore_axis_name="s",
        num_cores=_NC, num_subcores=_NS)

    @functools.partial(
        pl.kernel,
        out_type=jax.ShapeDtypeStruct((rows_total, 128), jnp.float32),
        mesh=mesh,
        compiler_params=pltpu.CompilerParams(
            needs_layout_passes=False, use_tc_tiling_on_sc=True),
        scratch_types=[
            pltpu.VMEM((ta_rows, 256), jnp.int32),
            pltpu.VMEM((tp_rows, 256), jnp.int32),
            pltpu.VMEM((2, _CROWS, 128), jnp.int32),
            pltpu.VMEM((2, _CROWS, 128), jnp.int32),
            pltpu.VMEM((2, _CROWS, 128), jnp.float32),
            pltpu.SemaphoreType.DMA,
            pltpu.SemaphoreType.DMA,
            pltpu.SemaphoreType.DMA,
            pltpu.SemaphoreType.DMA,
        ],
    )
    def edge_kernel(sa_hbm, sp_hbm, src_hbm, dst_hbm, out_hbm,
                    sa_v, sp_v, src_v, dst_v, out_v,
                    s_in0, s_in1, s_out0, s_out1):
        s_in = (s_in0, s_in1)
        s_out = (s_out0, s_out1)
        wid = lax.axis_index("s") * _NC + lax.axis_index("c")
        row_lo = (wid * base8 + jnp.minimum(wid, extra8)) * 8
        n_rows = (base8 + jnp.where(wid < extra8, 1, 0)) * 8
        last_off = row_lo + n_rows - _CROWS

        def off_of(c):
            return pl.multiple_of(
                jnp.minimum(row_lo + c * _CROWS, last_off), 8)

        def start_in(c, b):
            off = off_of(c)
            pltpu.async_copy(src_hbm.at[pl.ds(off, _CROWS), :],
                             src_v.at[b], s_in[b])
            pltpu.async_copy(dst_hbm.at[pl.ds(off, _CROWS), :],
                             dst_v.at[b], s_in[b])

        def wait_in(c, b):
            off = off_of(c)
            pltpu.make_async_copy(src_hbm.at[pl.ds(off, _CROWS), :],
                                  src_v.at[b], s_in[b]).wait()
            pltpu.make_async_copy(dst_hbm.at[pl.ds(off, _CROWS), :],
                                  dst_v.at[b], s_in[b]).wait()

        def wait_out(c, b):
            off = off_of(c)
            pltpu.make_async_copy(out_v.at[b],
                                  out_hbm.at[pl.ds(off, _CROWS), :],
                                  s_out[b]).wait()

        pltpu.sync_copy(sa_hbm, sa_v)
        pltpu.sync_copy(sp_hbm, sp_v)
        start_in(0, 0)
        start_in(1, 1)

        def pair(cc, carry):
            for b in (0, 1):
                c = cc * 2 + b

                @pl.when(c < chunks)
                def _():
                    wait_in(c, b)

                    @pl.when(c >= 2)
                    def _():
                        wait_out(c - 2, b)

                    @plsc.parallel_loop(0, _CROWS, unroll=4)
                    def _(r):
                        for j in range(8):
                            sl = pl.ds(j * _L, _L)
                            s = src_v[b, r, sl]
                            d = dst_v[b, r, sl]
                            ws = plsc.load_gather(sa_v, [s >> 9, s & 255])
                            wd = plsc.load_gather(sp_v, [d >> 9, d & 255])
                            fs = plsc.bitcast(
                                lax.shift_right_logical(
                                    ws, (s & 256) >> 4) << 16, jnp.float32)
                            fd = plsc.bitcast(
                                lax.shift_right_logical(
                                    wd, (d & 256) >> 4) << 16, jnp.float32)
                            out_v[b, r, sl] = fs + fd
                    pltpu.async_copy(out_v.at[b],
                                     out_hbm.at[pl.ds(off_of(c), _CROWS), :],
                                     s_out[b])

                    @pl.when(c + 2 < chunks)
                    def _():
                        start_in(c + 2, b)
            return carry

        lax.fori_loop(0, (chunks + 1) // 2, pair, 0)
        wait_out(chunks - 1, (chunks - 1) % 2)
        wait_out(chunks - 2, (chunks - 2) % 2)

    return edge_kernel


def kernel(author_x, paper_x, src_index, dst_index,
           W_author, b_author, W_paper, b_paper, W_scorer, b_scorer):
    d = author_x.shape[1]
    e = src_index.shape[0]

    w1 = W_scorer[0, :d]
    w2 = W_scorer[0, d:]
    v_a = (W_author.T @ w1).reshape(d, 1)
    v_p = (W_paper.T @ w2).reshape(d, 1)
    const = (b_scorer[0] + b_author @ w1 + b_paper @ w2).reshape(1)

    sa_pk, sp_pk = _encode(author_x.T, paper_x.T, v_a, v_p, const)

    src = src_index.astype(jnp.int32)
    dst = dst_index.astype(jnp.int32)
    min_e = _KE * _NW
    e_pad = -(-max(e, min_e) // 1024) * 1024
    if e_pad != e:
        pad = e_pad - e
        src = jnp.concatenate([src, jnp.zeros((pad,), jnp.int32)])
        dst = jnp.concatenate([dst, jnp.zeros((pad,), jnp.int32)])
    rows = e_pad // 128

    edge_kernel = _make_edge_kernel(rows, sa_pk.shape[0], sp_pk.shape[0])
    out2 = edge_kernel(sa_pk, sp_pk,
                       src.reshape(rows, 128), dst.reshape(rows, 128))
    out = out2.reshape(e_pad)
    return out[:e] if e_pad != e else out


# speedup vs baseline: 1.1834x; 1.1834x over previous
import functools

import jax
import jax.numpy as jnp
from jax import lax
from jax.experimental import pallas as pl
from jax.experimental.pallas import tpu as pltpu
from jax.experimental.pallas import tpu_sc as plsc

_NC = 2
_NS = 16
_NW = _NC * _NS
_L = 16

_GB = 4096


def _rne_bf16_bits(x):
    b = lax.bitcast_convert_type(x, jnp.int32)
    r = b + 0x7FFF + (lax.shift_right_logical(b, 16) & 1)
    return lax.shift_right_logical(r, 16)


def _pack8(s):
    rows = [
        _rne_bf16_bits(s[:, 512 * j:512 * j + 256])
        | (_rne_bf16_bits(s[:, 512 * j + 256:512 * j + 512]) << 16)
        for j in range(8)
    ]
    return jnp.concatenate(rows, axis=0)


def _encode_body(ax_ref, px_ref, va_ref, vp_ref, c_ref, oa_ref, op_ref):
    asum = jnp.sum(ax_ref[...] * va_ref[...], axis=0, keepdims=True)
    oa_ref[...] = _pack8(asum)
    psum = jnp.sum(px_ref[...] * vp_ref[...], axis=0, keepdims=True) + c_ref[0]
    op_ref[...] = _pack8(psum)


def _encode(axt, pxt, va, vp, const):
    d, na = axt.shape
    npp = pxt.shape[1]
    grid = -(-max(na, npp) // _GB)
    nba = -(-na // _GB) - 1
    nbp = -(-npp // _GB) - 1

    return pl.pallas_call(
        _encode_body,
        grid=(grid,),
        out_shape=[
            jax.ShapeDtypeStruct((8 * grid, 256), jnp.int32),
            jax.ShapeDtypeStruct((8 * grid, 256), jnp.int32),
        ],
        in_specs=[
            pl.BlockSpec((d, _GB), lambda g: (0, jnp.minimum(g, nba))),
            pl.BlockSpec((d, _GB), lambda g: (0, jnp.minimum(g, nbp))),
            pl.BlockSpec((d, 1), lambda g: (0, 0)),
            pl.BlockSpec((d, 1), lambda g: (0, 0)),
            pl.BlockSpec(memory_space=pltpu.SMEM),
        ],
        out_specs=[
            pl.BlockSpec((8, 256), lambda g: (g, 0)),
            pl.BlockSpec((8, 256), lambda g: (g, 0)),
        ],
    )(axt, pxt, va, vp, const)


_CROWS = 32
_KE = _CROWS * 128


def _make_edge_kernel(rows_total, ta_rows, tp_rows):
    rows8 = rows_total // 8
    base8 = rows8 // _NW
    extra8 = rows8 - base8 * _NW
    chunks = -(-(base8 + 1) * 8 // _CROWS)
    steps = _KE // _L
    mesh = plsc.VectorSubcoreMesh(
        core_axis_name="c", subcore_axis_name="s",
        num_cores=_NC, num_subcores=_NS)

    @functools.partial(
        pl.kernel,
        out_type=jax.ShapeDtypeStruct((rows_total, 128), jnp.float32),
        mesh=mesh,
        compiler_params=pltpu.CompilerParams(
            needs_layout_passes=False, use_tc_tiling_on_sc=True),
        scratch_types=[
            pltpu.VMEM((ta_rows, 256), jnp.int32),
            pltpu.VMEM((tp_rows, 256), jnp.int32),
            pltpu.VMEM((2, _CROWS, 128), jnp.int32),
            pltpu.VMEM((2, _CROWS, 128), jnp.int32),
            pltpu.VMEM((2, _CROWS, 128), jnp.float32),
            pltpu.SemaphoreType.DMA,
            pltpu.SemaphoreType.DMA,
            pltpu.SemaphoreType.DMA,
            pltpu.SemaphoreType.DMA,
        ],
    )
    def edge_kernel(sa_hbm, sp_hbm, src_hbm, dst_hbm, out_hbm,
                    sa_v, sp_v, src_v, dst_v, out_v,
                    s_in0, s_in1, s_out0, s_out1):
        s_in = (s_in0, s_in1)
        s_out = (s_out0, s_out1)
        wid = lax.axis_index("s") * _NC + lax.axis_index("c")
        row_lo = (wid * base8 + jnp.minimum(wid, extra8)) * 8
        n_rows = (base8 + jnp.where(wid < extra8, 1, 0)) * 8
        last_off = row_lo + n_rows - _CROWS

        def off_of(c):
            return pl.multiple_of(
                jnp.minimum(row_lo + c * _CROWS, last_off), 8)

        def start_in(c, b):
            off = off_of(c)
            pltpu.async_copy(src_hbm.at[pl.ds(off, _CROWS), :],
                             src_v.at[b], s_in[b])
            pltpu.async_copy(dst_hbm.at[pl.ds(off, _CROWS), :],
                             dst_v.at[b], s_in[b])

        def wait_in(c, b):
            off = off_of(c)
            pltpu.make_async_copy(src_hbm.at[pl.ds(off, _CROWS), :],
                                  src_v.at[b], s_in[b]).wait()
            pltpu.make_async_copy(dst_hbm.at[pl.ds(off, _CROWS), :],
                                  dst_v.at[b], s_in[b]).wait()

        def wait_out(c, b):
            off = off_of(c)
            pltpu.make_async_copy(out_v.at[b],
                                  out_hbm.at[pl.ds(off, _CROWS), :],
                                  s_out[b]).wait()

        pltpu.sync_copy(sa_hbm, sa_v)
        pltpu.sync_copy(sp_hbm, sp_v)
        start_in(0, 0)
        start_in(1, 1)

        def pair(cc, carry):
            for b in (0, 1):
                c = cc * 2 + b

                @pl.when(c < chunks)
                def _():
                    wait_in(c, b)

                    @pl.when(c >= 2)
                    def _():
                        wait_out(c - 2, b)

                    @plsc.parallel_loop(0, _CROWS, unroll=1)
                    def _(r):
                        for j in range(8):
                            sl = pl.ds(j * _L, _L)
                            s = src_v[b, r, sl]
                            d = dst_v[b, r, sl]
                            ws = plsc.load_gather(sa_v, [s >> 9, s & 255])
                            wd = plsc.load_gather(sp_v, [d >> 9, d & 255])
                            fs = plsc.bitcast(
                                lax.shift_right_logical(
                                    ws, (s & 256) >> 4) << 16, jnp.float32)
                            fd = plsc.bitcast(
                                lax.shift_right_logical(
                                    wd, (d & 256) >> 4) << 16, jnp.float32)
                            out_v[b, r, sl] = fs + fd
                    pltpu.async_copy(out_v.at[b],
                                     out_hbm.at[pl.ds(off_of(c), _CROWS), :],
                                     s_out[b])

                    @pl.when(c + 2 < chunks)
                    def _():
                        start_in(c + 2, b)
            return carry

        lax.fori_loop(0, (chunks + 1) // 2, pair, 0)
        wait_out(chunks - 1, (chunks - 1) % 2)
        wait_out(chunks - 2, (chunks - 2) % 2)

    return edge_kernel


def kernel(author_x, paper_x, src_index, dst_index,
           W_author, b_author, W_paper, b_paper, W_scorer, b_scorer):
    d = author_x.shape[1]
    e = src_index.shape[0]

    w1 = W_scorer[0, :d]
    w2 = W_scorer[0, d:]
    v_a = (W_author.T @ w1).reshape(d, 1)
    v_p = (W_paper.T @ w2).reshape(d, 1)
    const = (b_scorer[0] + b_author @ w1 + b_paper @ w2).reshape(1)

    sa_pk, sp_pk = _encode(author_x.T, paper_x.T, v_a, v_p, const)

    src = src_index.astype(jnp.int32)
    dst = dst_index.astype(jnp.int32)
    min_e = _KE * _NW
    e_pad = -(-max(e, min_e) // 1024) * 1024
    if e_pad != e:
        pad = e_pad - e
        src = jnp.concatenate([src, jnp.zeros((pad,), jnp.int32)])
        dst = jnp.concatenate([dst, jnp.zeros((pad,), jnp.int32)])
    rows = e_pad // 128

    edge_kernel = _make_edge_kernel(rows, sa_pk.shape[0], sp_pk.shape[0])
    out2 = edge_kernel(sa_pk, sp_pk,
                       src.reshape(rows, 128), dst.reshape(rows, 128))
    out = out2.reshape(e_pad)
    return out[:e] if e_pad != e else out


# speedup vs baseline: 1.3359x; 1.1288x over previous
import functools

import jax
import jax.numpy as jnp
from jax import lax
from jax.experimental import pallas as pl
from jax.experimental.pallas import tpu as pltpu
from jax.experimental.pallas import tpu_sc as plsc

_NC = 2
_NS = 16
_NW = _NC * _NS
_L = 16

_GB = 20480


def _rne_bf16_bits(x):
    b = lax.bitcast_convert_type(x, jnp.int32)
    r = b + 0x7FFF + (lax.shift_right_logical(b, 16) & 1)
    return lax.shift_right_logical(r, 16)


def _packrows(s):
    rows = [
        _rne_bf16_bits(s[:, 512 * j:512 * j + 256])
        | (_rne_bf16_bits(s[:, 512 * j + 256:512 * j + 512]) << 16)
        for j in range(s.shape[1] // 512)
    ]
    return jnp.concatenate(rows, axis=0)


def _encode_body(ax_ref, px_ref, wa_ref, ba_ref, wp_ref, bp_ref, ws_ref,
                 bs_ref, oa_ref, op_ref):
    d = ax_ref.shape[0]
    va = [sum(ws_ref[0, k] * wa_ref[k, i] for k in range(d))
          for i in range(d)]
    vp = [sum(ws_ref[0, d + k] * wp_ref[k, i] for k in range(d))
          for i in range(d)]
    cc = (bs_ref[0]
          + sum(ws_ref[0, k] * ba_ref[k] for k in range(d))
          + sum(ws_ref[0, d + k] * bp_ref[k] for k in range(d)))
    xa = ax_ref[...]
    xp = px_ref[...]
    asum = sum(va[i] * xa[i:i + 1, :] for i in range(d))
    oa_ref[...] = _packrows(asum)
    psum = sum(vp[i] * xp[i:i + 1, :] for i in range(d)) + cc
    op_ref[...] = _packrows(psum)


def _encode(axt, pxt, w_a, b_a, w_p, b_p, w_s, b_s):
    d, na = axt.shape
    npp = pxt.shape[1]
    grid = -(-max(na, npp) // _GB)
    nba = -(-na // _GB) - 1
    nbp = -(-npp // _GB) - 1
    orows = _GB // 512

    smem = pl.BlockSpec(memory_space=pltpu.SMEM)
    return pl.pallas_call(
        _encode_body,
        grid=(grid,),
        out_shape=[
            jax.ShapeDtypeStruct((orows * grid, 256), jnp.int32),
            jax.ShapeDtypeStruct((orows * grid, 256), jnp.int32),
        ],
        in_specs=[
            pl.BlockSpec((d, _GB), lambda g: (0, jnp.minimum(g, nba))),
            pl.BlockSpec((d, _GB), lambda g: (0, jnp.minimum(g, nbp))),
            smem, smem, smem, smem, smem, smem,
        ],
        out_specs=[
            pl.BlockSpec((orows, 256), lambda g: (g, 0)),
            pl.BlockSpec((orows, 256), lambda g: (g, 0)),
        ],
    )(axt, pxt, w_a, b_a, w_p, b_p, w_s, b_s)


_CROWS = 32
_KE = _CROWS * 128


def _make_edge_kernel(rows_total, ta_rows, tp_rows):
    rows8 = rows_total // 8
    base8 = rows8 // _NW
    extra8 = rows8 - base8 * _NW
    chunks = -(-(base8 + 1) * 8 // _CROWS)
    steps = _KE // _L
    mesh = plsc.VectorSubcoreMesh(
        core_axis_name="c", subcore_axis_name="s",
        num_cores=_NC, num_subcores=_NS)

    @functools.partial(
        pl.kernel,
        out_type=jax.ShapeDtypeStruct((rows_total, 128), jnp.float32),
        mesh=mesh,
        compiler_params=pltpu.CompilerParams(
            needs_layout_passes=False, use_tc_tiling_on_sc=True),
        scratch_types=[
            pltpu.VMEM((ta_rows, 256), jnp.int32),
            pltpu.VMEM((tp_rows, 256), jnp.int32),
            pltpu.VMEM((2, _CROWS, 128), jnp.int32),
            pltpu.VMEM((2, _CROWS, 128), jnp.int32),
            pltpu.VMEM((2, _CROWS, 128), jnp.float32),
            pltpu.SemaphoreType.DMA,
            pltpu.SemaphoreType.DMA,
            pltpu.SemaphoreType.DMA,
            pltpu.SemaphoreType.DMA,
        ],
    )
    def edge_kernel(sa_hbm, sp_hbm, src_hbm, dst_hbm, out_hbm,
                    sa_v, sp_v, src_v, dst_v, out_v,
                    s_in0, s_in1, s_out0, s_out1):
        s_in = (s_in0, s_in1)
        s_out = (s_out0, s_out1)
        wid = lax.axis_index("s") * _NC + lax.axis_index("c")
        row_lo = (wid * base8 + jnp.minimum(wid, extra8)) * 8
        n_rows = (base8 + jnp.where(wid < extra8, 1, 0)) * 8
        last_off = row_lo + n_rows - _CROWS

        def off_of(c):
            return pl.multiple_of(
                jnp.minimum(row_lo + c * _CROWS, last_off), 8)

        def start_in(c, b):
            off = off_of(c)
            pltpu.async_copy(src_hbm.at[pl.ds(off, _CROWS), :],
                             src_v.at[b], s_in[b])
            pltpu.async_copy(dst_hbm.at[pl.ds(off, _CROWS), :],
                             dst_v.at[b], s_in[b])

        def wait_in(c, b):
            off = off_of(c)
            pltpu.make_async_copy(src_hbm.at[pl.ds(off, _CROWS), :],
                                  src_v.at[b], s_in[b]).wait()
            pltpu.make_async_copy(dst_hbm.at[pl.ds(off, _CROWS), :],
                                  dst_v.at[b], s_in[b]).wait()

        def wait_out(c, b):
            off = off_of(c)
            pltpu.make_async_copy(out_v.at[b],
                                  out_hbm.at[pl.ds(off, _CROWS), :],
                                  s_out[b]).wait()

        pltpu.sync_copy(sa_hbm, sa_v)
        pltpu.sync_copy(sp_hbm, sp_v)
        start_in(0, 0)
        start_in(1, 1)

        def pair(cc, carry):
            for b in (0, 1):
                c = cc * 2 + b

                @pl.when(c < chunks)
                def _():
                    wait_in(c, b)

                    @pl.when(c >= 2)
                    def _():
                        wait_out(c - 2, b)

                    @plsc.parallel_loop(0, _CROWS, unroll=1)
                    def _(r):
                        for j in range(8):
                            sl = pl.ds(j * _L, _L)
                            s = src_v[b, r, sl]
                            d = dst_v[b, r, sl]
                            ws = plsc.load_gather(sa_v, [s >> 9, s & 255])
                            wd = plsc.load_gather(sp_v, [d >> 9, d & 255])
                            fs = plsc.bitcast(
                                lax.shift_right_logical(
                                    ws, (s & 256) >> 4) << 16, jnp.float32)
                            fd = plsc.bitcast(
                                lax.shift_right_logical(
                                    wd, (d & 256) >> 4) << 16, jnp.float32)
                            out_v[b, r, sl] = fs + fd
                    pltpu.async_copy(out_v.at[b],
                                     out_hbm.at[pl.ds(off_of(c), _CROWS), :],
                                     s_out[b])

                    @pl.when(c + 2 < chunks)
                    def _():
                        start_in(c + 2, b)
            return carry

        lax.fori_loop(0, (chunks + 1) // 2, pair, 0)
        wait_out(chunks - 1, (chunks - 1) % 2)
        wait_out(chunks - 2, (chunks - 2) % 2)

    return edge_kernel


def kernel(author_x, paper_x, src_index, dst_index,
           W_author, b_author, W_paper, b_paper, W_scorer, b_scorer):
    e = src_index.shape[0]

    sa_pk, sp_pk = _encode(author_x.T, paper_x.T, W_author, b_author,
                           W_paper, b_paper, W_scorer, b_scorer)

    src = src_index.astype(jnp.int32)
    dst = dst_index.astype(jnp.int32)
    min_e = _KE * _NW
    e_pad = -(-max(e, min_e) // 1024) * 1024
    if e_pad != e:
        pad = e_pad - e
        src = jnp.concatenate([src, jnp.zeros((pad,), jnp.int32)])
        dst = jnp.concatenate([dst, jnp.zeros((pad,), jnp.int32)])
    rows = e_pad // 128

    edge_kernel = _make_edge_kernel(rows, sa_pk.shape[0], sp_pk.shape[0])
    out2 = edge_kernel(sa_pk, sp_pk,
                       src.reshape(rows, 128), dst.reshape(rows, 128))
    out = out2.reshape(e_pad)
    return out[:e] if e_pad != e else out
